# baseline (speedup 1.0000x reference)
import functools

import jax
import jax.numpy as jnp
from jax import lax
from jax.experimental import pallas as pl
from jax.experimental.pallas import tpu as pltpu
from jax.experimental.pallas import tpu_sc as plsc

N = 10000
K = 16
B = 8
NHID = 128
NCLS = 40

NPAD = 10240
RBLK = 128
NRB = NPAD // RBLK
CTILE = 512
NTIL = NPAD // CTILE
PBLK = 512
NPB = NPAD // PBLK

_PREC = lax.Precision.HIGHEST
_BIGI = 1 << 30


def _knn_body(tlo_ref, thi_ref, posT_ref, batT_ref, posR_ref, batR_ref,
              idx_ref, d_ref):
    i = pl.program_id(0)
    tlo = tlo_ref[i]
    thi = thi_ref[i]

    xr = posR_ref[...]
    x2r = jnp.sum(xr * xr, axis=1, keepdims=True)
    br = batR_ref[...][:, 0:1]

    def fill(t, carry):
        c0 = pl.multiple_of(t * CTILE, CTILE)
        pc = posT_ref[:, pl.ds(c0, CTILE)]
        dot = lax.dot_general(xr, pc, (((1,), (0,)), ((), ())),
                              preferred_element_type=jnp.float32,
                              precision=lax.Precision.DEFAULT)
        x2c = jnp.sum(pc * pc, axis=0, keepdims=True)
        d2 = (x2r + x2c) - 2.0 * dot
        bc = batT_ref[0:1, pl.ds(c0, CTILE)]
        d2 = jnp.where(br != bc, jnp.inf, d2)
        d_ref[:, pl.ds(c0, CTILE)] = d2
        return carry

    lax.fori_loop(tlo, thi, fill, 0)

    NRANK = 4

    def extract4(vprev, iprev):
        def tbody(t, carry):
            mv, miv = carry
            c0 = pl.multiple_of(t * CTILE, CTILE)
            d = d_ref[:, pl.ds(c0, CTILE)]
            colv = lax.broadcasted_iota(jnp.int32, (RBLK, CTILE), 1) + t * CTILE
            valid = (d > vprev) | ((d == vprev) & (colv > iprev))
            dm = jnp.where(valid, d, jnp.inf)
            tv, ti = [], []
            for r in range(NRANK):
                tmin = jnp.min(dm, axis=1, keepdims=True)
                targ = jnp.min(jnp.where(dm == tmin, colv, _BIGI), axis=1,
                               keepdims=True)
                tv.append(tmin)
                ti.append(targ)
                if r < NRANK - 1:
                    dm = jnp.where((dm == tmin) & (colv == targ), jnp.inf, dm)
            cv = jnp.concatenate([mv] + tv, axis=1)
            ci = jnp.concatenate([miv] + ti, axis=1)
            nv, ni = [], []
            for r in range(NRANK):
                m = jnp.min(cv, axis=1, keepdims=True)
                mi = jnp.min(jnp.where(cv == m, ci, _BIGI), axis=1,
                             keepdims=True)
                nv.append(m)
                ni.append(mi)
                if r < NRANK - 1:
                    cv = jnp.where((cv == m) & (ci == mi), jnp.inf, cv)
            return (jnp.concatenate(nv, axis=1), jnp.concatenate(ni, axis=1))

        m0 = jnp.full((RBLK, NRANK), jnp.inf, jnp.float32)
        i0 = jnp.zeros((RBLK, NRANK), jnp.int32)
        return lax.fori_loop(tlo, thi, tbody, (m0, i0))

    vprev = jnp.full((RBLK, 1), -jnp.inf, jnp.float32)
    iprev = jnp.full((RBLK, 1), -1, jnp.int32)
    outs = []
    for _ in range(K // NRANK):
        mv, miv = extract4(vprev, iprev)
        outs.append(miv)
        vprev = mv[:, NRANK - 1:NRANK]
        iprev = miv[:, NRANK - 1:NRANK]
    idx_ref[...] = jnp.concatenate(outs, axis=1)


def _knn_call(tlo, thi, posT, batT, posR, batR):
    return pl.pallas_call(
        _knn_body,
        grid=(NRB,),
        in_specs=[
            pl.BlockSpec(memory_space=pltpu.SMEM),
            pl.BlockSpec(memory_space=pltpu.SMEM),
            pl.BlockSpec((8, NPAD), lambda i: (0, 0)),
            pl.BlockSpec((8, NPAD), lambda i: (0, 0)),
            pl.BlockSpec((RBLK, 8), lambda i: (i, 0)),
            pl.BlockSpec((RBLK, 8), lambda i: (i, 0)),
        ],
        out_specs=pl.BlockSpec((RBLK, K), lambda i: (i, 0)),
        out_shape=jax.ShapeDtypeStruct((NPAD, K), jnp.int32),
        scratch_shapes=[pltpu.VMEM((RBLK, NPAD), jnp.float32)],
    )(tlo, thi, posT, batT, posR, batR)


def _proj_body(x_ref, wd_ref, wb_ref, b_ref, a_ref, v_ref):
    x = x_ref[...]
    a_ref[...] = lax.dot_general(x, wd_ref[...], (((1,), (0,)), ((), ())),
                                 preferred_element_type=jnp.float32,
                                 precision=_PREC) + b_ref[0:1, :]
    v_ref[...] = lax.dot_general(x, wb_ref[...], (((1,), (0,)), ((), ())),
                                 preferred_element_type=jnp.float32,
                                 precision=_PREC)


def _proj_call(x, wd, wb, b):
    f = x.shape[1]
    return pl.pallas_call(
        _proj_body,
        grid=(NPB,),
        in_specs=[
            pl.BlockSpec((PBLK, f), lambda i: (i, 0)),
            pl.BlockSpec((f, NHID), lambda i: (0, 0)),
            pl.BlockSpec((f, NHID), lambda i: (0, 0)),
            pl.BlockSpec((8, NHID), lambda i: (0, 0)),
        ],
        out_specs=[
            pl.BlockSpec((PBLK, NHID), lambda i: (i, 0)),
            pl.BlockSpec((PBLK, NHID), lambda i: (i, 0)),
        ],
        out_shape=[
            jax.ShapeDtypeStruct((NPAD, NHID), jnp.float32),
            jax.ShapeDtypeStruct((NPAD, NHID), jnp.float32),
        ],
    )(x, wd, wb, b)


def _comb_body(a_ref, g_ref, wd_ref, wb_ref, b_ref, a2_ref, v2_ref):
    h = jnp.maximum(a_ref[...] + g_ref[...], 0.0)
    a2_ref[...] = lax.dot_general(h, wd_ref[...], (((1,), (0,)), ((), ())),
                                  preferred_element_type=jnp.float32,
                                  precision=_PREC) + b_ref[0:1, :]
    v2_ref[...] = lax.dot_general(h, wb_ref[...], (((1,), (0,)), ((), ())),
                                  preferred_element_type=jnp.float32,
                                  precision=_PREC)


def _comb_call(a, g, wd, wb, b):
    return pl.pallas_call(
        _comb_body,
        grid=(NPB,),
        in_specs=[
            pl.BlockSpec((PBLK, NHID), lambda i: (i, 0)),
            pl.BlockSpec((PBLK, NHID), lambda i: (i, 0)),
            pl.BlockSpec((NHID, NHID), lambda i: (0, 0)),
            pl.BlockSpec((NHID, NHID), lambda i: (0, 0)),
            pl.BlockSpec((8, NHID), lambda i: (0, 0)),
        ],
        out_specs=[
            pl.BlockSpec((PBLK, NHID), lambda i: (i, 0)),
            pl.BlockSpec((PBLK, NHID), lambda i: (i, 0)),
        ],
        out_shape=[
            jax.ShapeDtypeStruct((NPAD, NHID), jnp.float32),
            jax.ShapeDtypeStruct((NPAD, NHID), jnp.float32),
        ],
    )(a, g, wd, wb, b)


def _final_body(a_ref, g_ref, batR_ref, wc_ref, bc_ref, out_ref, pool_ref):
    i = pl.program_id(0)

    @pl.when(i == 0)
    def _():
        pool_ref[...] = jnp.full((B, NHID), -jnp.inf, jnp.float32)

    h = jnp.maximum(a_ref[...] + g_ref[...], 0.0)
    bat = batR_ref[...][:, 0:1]
    for b in range(B):
        mb = jnp.where(bat == b, h, -jnp.inf)
        pb = jnp.max(mb, axis=0, keepdims=True)
        pool_ref[pl.ds(b, 1), :] = jnp.maximum(pool_ref[pl.ds(b, 1), :], pb)

    @pl.when(i == NPB - 1)
    def _():
        out_ref[...] = lax.dot_general(
            pool_ref[...], wc_ref[...], (((1,), (0,)), ((), ())),
            preferred_element_type=jnp.float32,
            precision=_PREC) + bc_ref[0:1, :]


def _final_call(a2, g2, batR, wc, bc):
    return pl.pallas_call(
        _final_body,
        grid=(NPB,),
        in_specs=[
            pl.BlockSpec((PBLK, NHID), lambda i: (i, 0)),
            pl.BlockSpec((PBLK, NHID), lambda i: (i, 0)),
            pl.BlockSpec((PBLK, 8), lambda i: (i, 0)),
            pl.BlockSpec((NHID, NCLS), lambda i: (0, 0)),
            pl.BlockSpec((8, NCLS), lambda i: (0, 0)),
        ],
        out_specs=pl.BlockSpec((B, NCLS), lambda i: (0, 0)),
        out_shape=jax.ShapeDtypeStruct((B, NCLS), jnp.float32),
        scratch_shapes=[pltpu.VMEM((B, NHID), jnp.float32)],
    )(a2, g2, batR, wc, bc)


_NW = 32
_RPW = NPAD // _NW
_CH = 8
_NCH = _RPW // _CH


def _gmax_body(v_hbm, idx_hbm, out_hbm, idx_v0, idx_v1, rows_v0, rows_v1,
               ob_v, sem0, sem1):
    wid = lax.axis_index("s") * 2 + lax.axis_index("c")
    base = wid * _RPW

    def reduce_store(rows_v, nb):
        for n in range(_CH):
            for fc in range(NHID // 16):
                acc = rows_v[n * K, pl.ds(fc * 16, 16)]
                for j in range(1, K):
                    acc = jnp.maximum(acc, rows_v[n * K + j, pl.ds(fc * 16, 16)])
                ob_v[n, pl.ds(fc * 16, 16)] = acc
        pltpu.sync_copy(ob_v, out_hbm.at[pl.ds(pl.multiple_of(nb, 8), _CH)])

    def pair(p, carry):
        nb0 = base + (2 * p) * _CH
        nb1 = nb0 + _CH
        pltpu.sync_copy(idx_hbm.at[pl.ds(pl.multiple_of(nb0 * K, 128), _CH * K)],
                        idx_v0)
        pltpu.sync_copy(idx_hbm.at[pl.ds(pl.multiple_of(nb1 * K, 128), _CH * K)],
                        idx_v1)
        h0 = pltpu.async_copy(v_hbm.at[idx_v0], rows_v0, sem0)
        h1 = pltpu.async_copy(v_hbm.at[idx_v1], rows_v1, sem1)
        h0.wait()
        reduce_store(rows_v0, nb0)
        h1.wait()
        reduce_store(rows_v1, nb1)
        return carry

    lax.fori_loop(0, _NCH // 2, pair, 0)


_gmax_call = pl.kernel(
    _gmax_body,
    out_type=jax.ShapeDtypeStruct((NPAD, NHID), jnp.float32),
    mesh=plsc.VectorSubcoreMesh(core_axis_name="c", subcore_axis_name="s"),
    scratch_types=[
        pltpu.VMEM((_CH * K,), jnp.int32),
        pltpu.VMEM((_CH * K,), jnp.int32),
        pltpu.VMEM((_CH * K, NHID), jnp.float32),
        pltpu.VMEM((_CH * K, NHID), jnp.float32),
        pltpu.VMEM((_CH, NHID), jnp.float32),
        pltpu.SemaphoreType.DMA,
        pltpu.SemaphoreType.DMA,
    ],
)


def kernel(pos, batch, W1, b1, W2, b2, Wc, bc):
    batch = batch.astype(jnp.int32)

    posp = jnp.zeros((NPAD, 8), jnp.float32).at[:N, :3].set(pos)
    batp = jnp.full((NPAD,), B, jnp.int32).at[:N].set(batch)
    posT = posp.T
    batT = jnp.broadcast_to(batp[None, :], (8, NPAD))
    batR = jnp.broadcast_to(batp[:, None], (NPAD, 8))

    seg = jnp.searchsorted(batch, jnp.arange(B + 1, dtype=jnp.int32),
                           side="left").astype(jnp.int32)
    starts = jnp.arange(NRB, dtype=jnp.int32) * RBLK
    blo = batch[jnp.minimum(starts, N - 1)]
    bhi = batch[jnp.minimum(starts + RBLK - 1, N - 1)]
    tlo = seg[blo] // CTILE
    thi = (seg[bhi + 1] + CTILE - 1) // CTILE
    allpad = starts >= N
    tlo = jnp.where(allpad, 0, tlo)
    thi = jnp.where(allpad, 0, thi)

    idx = _knn_call(tlo, thi, posT, batT, posp, batR)
    idxf = idx.reshape(-1)

    wd1 = jnp.zeros((8, NHID), jnp.float32).at[:3].set(W1[:3] - W1[3:])
    wb1 = jnp.zeros((8, NHID), jnp.float32).at[:3].set(W1[3:])
    b1b = jnp.broadcast_to(b1[None, :], (8, NHID))
    a1, v1 = _proj_call(posp, wd1, wb1, b1b)
    g1 = _gmax_call(v1, idxf)

    wd2 = W2[:NHID] - W2[NHID:]
    wb2 = W2[NHID:]
    b2b = jnp.broadcast_to(b2[None, :], (8, NHID))
    a2, v2 = _comb_call(a1, g1, wd2, wb2, b2b)
    g2 = _gmax_call(v2, idxf)

    bcb = jnp.broadcast_to(bc[None, :], (8, NCLS))
    return _final_call(a2, g2, batR, Wc, bcb)


# speedup vs baseline: 1.3591x; 1.3591x over previous
import functools

import jax
import jax.numpy as jnp
from jax import lax
from jax.experimental import pallas as pl
from jax.experimental.pallas import tpu as pltpu
from jax.experimental.pallas import tpu_sc as plsc

N = 10000
K = 16
B = 8
NHID = 128
NCLS = 40

NPAD = 10240
RBLK = 128
NRB = NPAD // RBLK
CTILE = 512
NTIL = NPAD // CTILE
PBLK = 512
NPB = NPAD // PBLK

_PREC = lax.Precision.HIGHEST
_BIGI = 1 << 30


def _knn_body(tlo_ref, thi_ref, posT_ref, batT_ref, posR_ref, batR_ref,
              idx_ref, d_ref):
    i = pl.program_id(0)
    tlo = tlo_ref[i]
    thi = thi_ref[i]

    xr = posR_ref[...]
    x2r = jnp.sum(xr * xr, axis=1, keepdims=True)
    br = batR_ref[...][:, 0:1]

    def fill(t, carry):
        c0 = pl.multiple_of(t * CTILE, CTILE)
        pc = posT_ref[:, pl.ds(c0, CTILE)]
        dot = lax.dot_general(xr, pc, (((1,), (0,)), ((), ())),
                              preferred_element_type=jnp.float32,
                              precision=lax.Precision.DEFAULT)
        x2c = jnp.sum(pc * pc, axis=0, keepdims=True)
        d2 = (x2r + x2c) - 2.0 * dot
        bc = batT_ref[0:1, pl.ds(c0, CTILE)]
        d2 = jnp.where(br != bc, jnp.inf, d2)
        d_ref[:, pl.ds(c0, CTILE)] = d2
        return carry

    lax.fori_loop(tlo, thi, fill, 0)

    def extract_one(vprev, iprev):
        def tbody(t, carry):
            m, mi = carry
            c0 = pl.multiple_of(t * CTILE, CTILE)
            d = d_ref[:, pl.ds(c0, CTILE)]
            colv = lax.broadcasted_iota(jnp.int32, (RBLK, CTILE), 1) + t * CTILE
            valid = (d > vprev) | ((d == vprev) & (colv > iprev))
            dm = jnp.where(valid, d, jnp.inf)
            tmin = jnp.min(dm, axis=1, keepdims=True)
            targ = jnp.min(jnp.where(dm == tmin, colv, _BIGI), axis=1,
                           keepdims=True)
            better = (tmin < m) | ((tmin == m) & (targ < mi))
            return (jnp.where(better, tmin, m), jnp.where(better, targ, mi))

        m0 = jnp.full((RBLK, 1), jnp.inf, jnp.float32)
        i0 = jnp.zeros((RBLK, 1), jnp.int32)
        return lax.fori_loop(tlo, thi, tbody, (m0, i0))

    vprev = jnp.full((RBLK, 1), -jnp.inf, jnp.float32)
    iprev = jnp.full((RBLK, 1), -1, jnp.int32)
    outs = []
    for _ in range(K):
        vprev, iprev = extract_one(vprev, iprev)
        outs.append(iprev)
    idx_ref[...] = jnp.concatenate(outs, axis=1)


def _knn_call(tlo, thi, posT, batT, posR, batR):
    return pl.pallas_call(
        _knn_body,
        grid=(NRB,),
        in_specs=[
            pl.BlockSpec(memory_space=pltpu.SMEM),
            pl.BlockSpec(memory_space=pltpu.SMEM),
            pl.BlockSpec((8, NPAD), lambda i: (0, 0)),
            pl.BlockSpec((8, NPAD), lambda i: (0, 0)),
            pl.BlockSpec((RBLK, 8), lambda i: (i, 0)),
            pl.BlockSpec((RBLK, 8), lambda i: (i, 0)),
        ],
        out_specs=pl.BlockSpec((RBLK, K), lambda i: (i, 0)),
        out_shape=jax.ShapeDtypeStruct((NPAD, K), jnp.int32),
        scratch_shapes=[pltpu.VMEM((RBLK, NPAD), jnp.float32)],
    )(tlo, thi, posT, batT, posR, batR)


def _proj_body(x_ref, wd_ref, wb_ref, b_ref, a_ref, v_ref):
    x = x_ref[...]
    a_ref[...] = lax.dot_general(x, wd_ref[...], (((1,), (0,)), ((), ())),
                                 preferred_element_type=jnp.float32,
                                 precision=_PREC) + b_ref[0:1, :]
    v_ref[...] = lax.dot_general(x, wb_ref[...], (((1,), (0,)), ((), ())),
                                 preferred_element_type=jnp.float32,
                                 precision=_PREC)


def _proj_call(x, wd, wb, b):
    f = x.shape[1]
    return pl.pallas_call(
        _proj_body,
        grid=(NPB,),
        in_specs=[
            pl.BlockSpec((PBLK, f), lambda i: (i, 0)),
            pl.BlockSpec((f, NHID), lambda i: (0, 0)),
            pl.BlockSpec((f, NHID), lambda i: (0, 0)),
            pl.BlockSpec((8, NHID), lambda i: (0, 0)),
        ],
        out_specs=[
            pl.BlockSpec((PBLK, NHID), lambda i: (i, 0)),
            pl.BlockSpec((PBLK, NHID), lambda i: (i, 0)),
        ],
        out_shape=[
            jax.ShapeDtypeStruct((NPAD, NHID), jnp.float32),
            jax.ShapeDtypeStruct((NPAD, NHID), jnp.float32),
        ],
    )(x, wd, wb, b)


def _comb_body(a_ref, g_ref, wd_ref, wb_ref, b_ref, a2_ref, v2_ref):
    h = jnp.maximum(a_ref[...] + g_ref[...], 0.0)
    a2_ref[...] = lax.dot_general(h, wd_ref[...], (((1,), (0,)), ((), ())),
                                  preferred_element_type=jnp.float32,
                                  precision=_PREC) + b_ref[0:1, :]
    v2_ref[...] = lax.dot_general(h, wb_ref[...], (((1,), (0,)), ((), ())),
                                  preferred_element_type=jnp.float32,
                                  precision=_PREC)


def _comb_call(a, g, wd, wb, b):
    return pl.pallas_call(
        _comb_body,
        grid=(NPB,),
        in_specs=[
            pl.BlockSpec((PBLK, NHID), lambda i: (i, 0)),
            pl.BlockSpec((PBLK, NHID), lambda i: (i, 0)),
            pl.BlockSpec((NHID, NHID), lambda i: (0, 0)),
            pl.BlockSpec((NHID, NHID), lambda i: (0, 0)),
            pl.BlockSpec((8, NHID), lambda i: (0, 0)),
        ],
        out_specs=[
            pl.BlockSpec((PBLK, NHID), lambda i: (i, 0)),
            pl.BlockSpec((PBLK, NHID), lambda i: (i, 0)),
        ],
        out_shape=[
            jax.ShapeDtypeStruct((NPAD, NHID), jnp.float32),
            jax.ShapeDtypeStruct((NPAD, NHID), jnp.float32),
        ],
    )(a, g, wd, wb, b)


def _final_body(a_ref, g_ref, batR_ref, wc_ref, bc_ref, out_ref, pool_ref):
    i = pl.program_id(0)

    @pl.when(i == 0)
    def _():
        pool_ref[...] = jnp.full((B, NHID), -jnp.inf, jnp.float32)

    h = jnp.maximum(a_ref[...] + g_ref[...], 0.0)
    bat = batR_ref[...][:, 0:1]
    for b in range(B):
        mb = jnp.where(bat == b, h, -jnp.inf)
        pb = jnp.max(mb, axis=0, keepdims=True)
        pool_ref[pl.ds(b, 1), :] = jnp.maximum(pool_ref[pl.ds(b, 1), :], pb)

    @pl.when(i == NPB - 1)
    def _():
        out_ref[...] = lax.dot_general(
            pool_ref[...], wc_ref[...], (((1,), (0,)), ((), ())),
            preferred_element_type=jnp.float32,
            precision=_PREC) + bc_ref[0:1, :]


def _final_call(a2, g2, batR, wc, bc):
    return pl.pallas_call(
        _final_body,
        grid=(NPB,),
        in_specs=[
            pl.BlockSpec((PBLK, NHID), lambda i: (i, 0)),
            pl.BlockSpec((PBLK, NHID), lambda i: (i, 0)),
            pl.BlockSpec((PBLK, 8), lambda i: (i, 0)),
            pl.BlockSpec((NHID, NCLS), lambda i: (0, 0)),
            pl.BlockSpec((8, NCLS), lambda i: (0, 0)),
        ],
        out_specs=pl.BlockSpec((B, NCLS), lambda i: (0, 0)),
        out_shape=jax.ShapeDtypeStruct((B, NCLS), jnp.float32),
        scratch_shapes=[pltpu.VMEM((B, NHID), jnp.float32)],
    )(a2, g2, batR, wc, bc)


_NW = 32
_RPW = NPAD // _NW
_CH = 8
_NCH = _RPW // _CH


def _gmax_body(v_hbm, idx_hbm, out_hbm, idx_v0, idx_v1, rows_v0, rows_v1,
               ob_v, sem0, sem1):
    wid = lax.axis_index("s") * 2 + lax.axis_index("c")
    base = wid * _RPW

    def reduce_store(rows_v, nb):
        for n in range(_CH):
            for fc in range(NHID // 16):
                acc = rows_v[n * K, pl.ds(fc * 16, 16)]
                for j in range(1, K):
                    acc = jnp.maximum(acc, rows_v[n * K + j, pl.ds(fc * 16, 16)])
                ob_v[n, pl.ds(fc * 16, 16)] = acc
        pltpu.sync_copy(ob_v, out_hbm.at[pl.ds(pl.multiple_of(nb, 8), _CH)])

    def pair(p, carry):
        nb0 = base + (2 * p) * _CH
        nb1 = nb0 + _CH
        pltpu.sync_copy(idx_hbm.at[pl.ds(pl.multiple_of(nb0 * K, 128), _CH * K)],
                        idx_v0)
        pltpu.sync_copy(idx_hbm.at[pl.ds(pl.multiple_of(nb1 * K, 128), _CH * K)],
                        idx_v1)
        h0 = pltpu.async_copy(v_hbm.at[idx_v0], rows_v0, sem0)
        h1 = pltpu.async_copy(v_hbm.at[idx_v1], rows_v1, sem1)
        h0.wait()
        reduce_store(rows_v0, nb0)
        h1.wait()
        reduce_store(rows_v1, nb1)
        return carry

    lax.fori_loop(0, _NCH // 2, pair, 0)


_gmax_call = pl.kernel(
    _gmax_body,
    out_type=jax.ShapeDtypeStruct((NPAD, NHID), jnp.float32),
    mesh=plsc.VectorSubcoreMesh(core_axis_name="c", subcore_axis_name="s"),
    scratch_types=[
        pltpu.VMEM((_CH * K,), jnp.int32),
        pltpu.VMEM((_CH * K,), jnp.int32),
        pltpu.VMEM((_CH * K, NHID), jnp.float32),
        pltpu.VMEM((_CH * K, NHID), jnp.float32),
        pltpu.VMEM((_CH, NHID), jnp.float32),
        pltpu.SemaphoreType.DMA,
        pltpu.SemaphoreType.DMA,
    ],
)


def kernel(pos, batch, W1, b1, W2, b2, Wc, bc):
    batch = batch.astype(jnp.int32)

    posp = jnp.zeros((NPAD, 8), jnp.float32).at[:N, :3].set(pos)
    batp = jnp.full((NPAD,), B, jnp.int32).at[:N].set(batch)
    posT = posp.T
    batT = jnp.broadcast_to(batp[None, :], (8, NPAD))
    batR = jnp.broadcast_to(batp[:, None], (NPAD, 8))

    seg = jnp.searchsorted(batch, jnp.arange(B + 1, dtype=jnp.int32),
                           side="left").astype(jnp.int32)
    starts = jnp.arange(NRB, dtype=jnp.int32) * RBLK
    blo = batch[jnp.minimum(starts, N - 1)]
    bhi = batch[jnp.minimum(starts + RBLK - 1, N - 1)]
    tlo = seg[blo] // CTILE
    thi = (seg[bhi + 1] + CTILE - 1) // CTILE
    allpad = starts >= N
    tlo = jnp.where(allpad, 0, tlo)
    thi = jnp.where(allpad, 0, thi)

    idx = _knn_call(tlo, thi, posT, batT, posp, batR)
    idxf = idx.reshape(-1)

    wd1 = jnp.zeros((8, NHID), jnp.float32).at[:3].set(W1[:3] - W1[3:])
    wb1 = jnp.zeros((8, NHID), jnp.float32).at[:3].set(W1[3:])
    b1b = jnp.broadcast_to(b1[None, :], (8, NHID))
    a1, v1 = _proj_call(posp, wd1, wb1, b1b)
    g1 = _gmax_call(v1, idxf)

    wd2 = W2[:NHID] - W2[NHID:]
    wb2 = W2[NHID:]
    b2b = jnp.broadcast_to(b2[None, :], (8, NHID))
    a2, v2 = _comb_call(a1, g1, wd2, wb2, b2b)
    g2 = _gmax_call(v2, idxf)

    bcb = jnp.broadcast_to(bc[None, :], (8, NCLS))
    return _final_call(a2, g2, batR, Wc, bcb)
